# Initial kernel scaffold; baseline (speedup 1.0000x reference)
import jax
import jax.numpy as jnp
from jax.experimental import pallas as pl


def kernel(x, edge_index, W, b, degree_table):
    raise NotImplementedError("write your pallas kernel here")


# speedup vs baseline: 6.6455x; 6.6455x over previous
import functools

import jax
import jax.numpy as jnp
from jax import lax
from jax.experimental import pallas as pl
from jax.experimental.pallas import tpu as pltpu
from jax.experimental.pallas import tpu_sc as plsc


NC = 2
NS = 16
L = 16


def _sc_degree_histogram(col, n_nodes):
  e_total = col.shape[0]
  nw = NC * NS
  assert e_total % (nw * L) == 0
  ec = e_total // nw
  npad = ((n_nodes + NS * L - 1) // (NS * L)) * (NS * L)
  rs = npad // NS

  mesh = plsc.VectorSubcoreMesh(core_axis_name="c", subcore_axis_name="s")

  @functools.partial(
      pl.kernel,
      out_type=jax.ShapeDtypeStruct((NC, npad), jnp.float32),
      mesh=mesh,
      compiler_params=pltpu.CompilerParams(needs_layout_passes=False),
      scratch_types=[
          pltpu.VMEM((ec,), jnp.int32),
          pltpu.VMEM((npad,), jnp.float32),
          pltpu.VMEM_SHARED((NS, npad), jnp.float32),
          pltpu.VMEM((NS, rs), jnp.float32),
          pltpu.VMEM((rs,), jnp.float32),
      ],
  )
  def hist_kernel(col_hbm, deg_hbm, idx_v, hist_v, shared, red_v, acc_v):
    cid = lax.axis_index("c")
    sid = lax.axis_index("s")
    wid = cid * NS + sid

    zeros = jnp.zeros((L,), jnp.float32)

    def zbody(i, _):
      hist_v[pl.ds(i * L, L)] = zeros
      return 0

    lax.fori_loop(0, npad // L, zbody, 0)

    pltpu.sync_copy(col_hbm.at[pl.ds(wid * ec, ec)], idx_v)

    ones = jnp.ones((L,), jnp.float32)

    def hbody(e, _):
      idx16 = idx_v[pl.ds(e * L, L)]
      plsc.addupdate_scatter(hist_v, [idx16], ones)
      return 0

    lax.fori_loop(0, ec // L, hbody, 0)

    pltpu.sync_copy(hist_v, shared.at[sid])
    plsc.subcore_barrier()

    pltpu.sync_copy(shared.at[:, pl.ds(sid * rs, rs)], red_v)

    def rbody(j, _):
      acc = red_v[0, pl.ds(j * L, L)]
      for k in range(1, NS):
        acc = acc + red_v[k, pl.ds(j * L, L)]
      acc_v[pl.ds(j * L, L)] = acc
      return 0

    lax.fori_loop(0, rs // L, rbody, 0)

    pltpu.sync_copy(acc_v, deg_hbm.at[cid, pl.ds(sid * rs, rs)])

  return hist_kernel, npad


def _tc_fuse(x, W, b2, table, d0, d1, block_rows):
  n, dfeat = x.shape
  emb = W.shape[0]
  vocab = table.shape[0]
  assert n % block_rows == 0
  grid = (n // block_rows,)

  def body(x_ref, w_ref, b_ref, t_ref, d0_ref, d1_ref, o_ref):
    feat = lax.dot_general(
        x_ref[...], w_ref[...],
        dimension_numbers=(((1,), (1,)), ((), ())),
        preferred_element_type=jnp.float32)
    deg = d0_ref[...] + d1_ref[...]
    di = jnp.minimum(deg.astype(jnp.int32), vocab - 1)
    iota = lax.broadcasted_iota(jnp.int32, (block_rows, vocab), 1)
    onehot = (di == iota).astype(jnp.float32)
    demb = lax.dot_general(
        onehot, t_ref[...],
        dimension_numbers=(((1,), (0,)), ((), ())),
        preferred_element_type=jnp.float32)
    o_ref[...] = feat + demb + b_ref[...]

  return pl.pallas_call(
      body,
      grid=grid,
      in_specs=[
          pl.BlockSpec((block_rows, dfeat), lambda i: (i, 0)),
          pl.BlockSpec((emb, dfeat), lambda i: (0, 0)),
          pl.BlockSpec((1, emb), lambda i: (0, 0)),
          pl.BlockSpec((vocab, emb), lambda i: (0, 0)),
          pl.BlockSpec((block_rows, 1), lambda i: (i, 0)),
          pl.BlockSpec((block_rows, 1), lambda i: (i, 0)),
      ],
      out_specs=pl.BlockSpec((block_rows, emb), lambda i: (i, 0)),
      out_shape=jax.ShapeDtypeStruct((n, emb), jnp.float32),
  )(x, W, b2, table, d0, d1)


def kernel(x, edge_index, W, b, degree_table):
  n = x.shape[0]
  col = edge_index[1].astype(jnp.int32)

  hist_kernel, _ = _sc_degree_histogram(col, n)
  deg2 = hist_kernel(col)

  d0 = deg2[0, :n].reshape(n, 1)
  d1 = deg2[1, :n].reshape(n, 1)
  b2 = b.reshape(1, -1)

  return _tc_fuse(x, W, b2, degree_table, d0, d1, block_rows=1000)


# speedup vs baseline: 9.4697x; 1.4250x over previous
import functools

import jax
import jax.numpy as jnp
from jax import lax
from jax.experimental import pallas as pl
from jax.experimental.pallas import tpu as pltpu
from jax.experimental.pallas import tpu_sc as plsc


NC = 2
NS = 16
L = 16
NW = NC * NS


def _sc_mesh():
  return plsc.VectorSubcoreMesh(core_axis_name="c", subcore_axis_name="s")


def _sc_degree_histogram(e_total, n_nodes):
  assert e_total % (NW * L) == 0
  ec = e_total // NW
  npad = ((n_nodes + NS * L - 1) // (NS * L)) * (NS * L)
  rs = npad // NS

  @functools.partial(
      pl.kernel,
      out_type=jax.ShapeDtypeStruct((NC * npad,), jnp.float32),
      mesh=_sc_mesh(),
      compiler_params=pltpu.CompilerParams(needs_layout_passes=False),
      scratch_types=[
          pltpu.VMEM((ec,), jnp.int32),
          pltpu.VMEM((npad,), jnp.float32),
          pltpu.VMEM_SHARED((NS, npad), jnp.float32),
          pltpu.VMEM((NS, rs), jnp.float32),
          pltpu.VMEM((rs,), jnp.float32),
      ],
  )
  def hist_kernel(ei_hbm, deg_hbm, idx_v, hist_v, shared, red_v, acc_v):
    cid = lax.axis_index("c")
    sid = lax.axis_index("s")
    wid = cid * NS + sid

    zeros = jnp.zeros((L,), jnp.float32)

    def zbody(i, _):
      hist_v[pl.ds(i * L, L)] = zeros
      return 0

    lax.fori_loop(0, npad // L, zbody, 0)

    pltpu.sync_copy(ei_hbm.at[pl.ds(e_total + wid * ec, ec)], idx_v)

    ones = jnp.ones((L,), jnp.float32)

    def hbody(e, _):
      idx16 = idx_v[pl.ds(e * L, L)]
      plsc.addupdate_scatter(hist_v, [idx16], ones)
      return 0

    lax.fori_loop(0, ec // L, hbody, 0)

    pltpu.sync_copy(hist_v, shared.at[sid])
    plsc.subcore_barrier()

    pltpu.sync_copy(shared.at[:, pl.ds(sid * rs, rs)], red_v)

    def rbody(j, _):
      acc = red_v[0, pl.ds(j * L, L)]
      for k in range(1, NS):
        acc = acc + red_v[k, pl.ds(j * L, L)]
      acc_v[pl.ds(j * L, L)] = acc
      return 0

    lax.fori_loop(0, rs // L, rbody, 0)

    pltpu.sync_copy(acc_v, deg_hbm.at[pl.ds(cid * npad + sid * rs, rs)])

  return hist_kernel, npad


def _tc_matmul(x, W, b2, block_rows):
  n, dfeat = x.shape
  emb = W.shape[0]
  assert n % block_rows == 0

  def body(x_ref, w_ref, b_ref, o_ref):
    o_ref[...] = lax.dot_general(
        x_ref[...], w_ref[...],
        dimension_numbers=(((1,), (1,)), ((), ())),
        preferred_element_type=jnp.float32) + b_ref[...]

  return pl.pallas_call(
      body,
      grid=(n // block_rows,),
      in_specs=[
          pl.BlockSpec((block_rows, dfeat), lambda i: (i, 0)),
          pl.BlockSpec((emb, dfeat), lambda i: (0, 0)),
          pl.BlockSpec((1, emb), lambda i: (0, 0)),
      ],
      out_specs=pl.BlockSpec((block_rows, emb), lambda i: (i, 0)),
      out_shape=jax.ShapeDtypeStruct((n, emb), jnp.float32),
  )(x, W, b2)


def _sc_combine(deg2, degree_table, feat, n_nodes):
  npad = deg2.shape[0] // NC
  vocab, emb = degree_table.shape
  rs = npad // NW
  last = n_nodes - (NW - 1) * rs
  gc = 64
  assert rs % gc == 0 and rs % L == 0 and 0 < last <= rs and last % 8 == 0

  @functools.partial(
      pl.kernel,
      out_type=jax.ShapeDtypeStruct((n_nodes, emb), jnp.float32),
      mesh=_sc_mesh(),
      compiler_params=pltpu.CompilerParams(needs_layout_passes=False),
      scratch_types=[
          pltpu.VMEM((rs,), jnp.float32),
          pltpu.VMEM((rs,), jnp.float32),
          pltpu.VMEM((rs,), jnp.int32),
          pltpu.VMEM((rs, emb), jnp.float32),
          pltpu.VMEM_SHARED((vocab, emb), jnp.float32),
          pltpu.SemaphoreType.DMA,
      ],
  )
  def combine_kernel(deg_hbm, tab_hbm, feat_hbm, out_hbm,
                     da_v, db_v, ii_v, acc_v, tab_sp, sem):
    cid = lax.axis_index("c")
    sid = lax.axis_index("s")
    wid = cid * NS + sid
    base = wid * rs

    @pl.when(sid == 0)
    def _():
      pltpu.sync_copy(tab_hbm, tab_sp)

    pltpu.sync_copy(deg_hbm.at[pl.ds(base, rs)], da_v)
    pltpu.sync_copy(deg_hbm.at[pl.ds(npad + base, rs)], db_v)

    @pl.when(wid < NW - 1)
    def _():
      pltpu.sync_copy(feat_hbm.at[pl.ds(base, rs)], acc_v)

    @pl.when(wid == NW - 1)
    def _():
      pltpu.sync_copy(feat_hbm.at[pl.ds(base, last)], acc_v.at[pl.ds(0, last)])

    def ibody(j, _):
      s = da_v[pl.ds(j * L, L)] + db_v[pl.ds(j * L, L)]
      ii_v[pl.ds(j * L, L)] = jnp.minimum(s.astype(jnp.int32), vocab - 1)
      return 0

    lax.fori_loop(0, rs // L, ibody, 0)

    plsc.subcore_barrier()

    for k in range(rs // gc):
      pltpu.async_copy(
          tab_sp.at[ii_v.at[pl.ds(k * gc, gc)]],
          acc_v.at[pl.ds(k * gc, gc)],
          sem, add=True).wait()

    @pl.when(wid < NW - 1)
    def _():
      pltpu.sync_copy(acc_v, out_hbm.at[pl.ds(base, rs)])

    @pl.when(wid == NW - 1)
    def _():
      pltpu.sync_copy(acc_v.at[pl.ds(0, last)], out_hbm.at[pl.ds(base, last)])

  return combine_kernel(deg2, degree_table, feat)


def kernel(x, edge_index, W, b, degree_table):
  n = x.shape[0]
  e_total = edge_index.shape[1]
  hist_kernel, _ = _sc_degree_histogram(e_total, n)
  deg2 = hist_kernel(edge_index.reshape(-1))
  feat = _tc_matmul(x, W, b.reshape(1, -1), block_rows=1000)
  return _sc_combine(deg2, degree_table, feat, n)


# speedup vs baseline: 10.8645x; 1.1473x over previous
import functools

import jax
import jax.numpy as jnp
from jax import lax
from jax.experimental import pallas as pl
from jax.experimental.pallas import tpu as pltpu
from jax.experimental.pallas import tpu_sc as plsc


NC = 2
NS = 16
L = 16
NW = NC * NS


def _sc_mesh():
  return plsc.VectorSubcoreMesh(core_axis_name="c", subcore_axis_name="s")


def _sc_degree_histogram(e_total, n_nodes):
  assert e_total % (NW * L) == 0
  ec = e_total // NW
  npad = ((n_nodes + NS * L - 1) // (NS * L)) * (NS * L)
  rs = npad // NS

  @functools.partial(
      pl.kernel,
      out_type=jax.ShapeDtypeStruct((NC * npad,), jnp.float32),
      mesh=_sc_mesh(),
      compiler_params=pltpu.CompilerParams(needs_layout_passes=False),
      scratch_types=[
          pltpu.VMEM((ec,), jnp.int32),
          pltpu.VMEM((npad,), jnp.float32),
          pltpu.VMEM_SHARED((NS, npad), jnp.float32),
          pltpu.VMEM((NS, rs), jnp.float32),
          pltpu.VMEM((rs,), jnp.float32),
          pltpu.SemaphoreType.DMA,
      ],
  )
  def hist_kernel(ei_hbm, deg_hbm, idx_v, hist_v, shared, red_v, acc_v, sem):
    cid = lax.axis_index("c")
    sid = lax.axis_index("s")
    wid = cid * NS + sid

    hu = 5
    eh = (ec // 2) // (L * hu) * (L * hu)
    cp0 = pltpu.async_copy(
        ei_hbm.at[pl.ds(e_total + wid * ec, eh)], idx_v.at[pl.ds(0, eh)], sem)
    cp1 = pltpu.async_copy(
        ei_hbm.at[pl.ds(e_total + wid * ec + eh, ec - eh)],
        idx_v.at[pl.ds(eh, ec - eh)], sem)

    zeros = jnp.zeros((L,), jnp.float32)
    zu = 4
    assert npad % (L * zu) == 0

    def zbody(i, _):
      for u in range(zu):
        hist_v[pl.ds((i * zu + u) * L, L)] = zeros
      return 0

    lax.fori_loop(0, npad // (L * zu), zbody, 0)

    ones = jnp.ones((L,), jnp.float32)
    assert eh % (L * hu) == 0 and ec % (L * hu) == 0

    def hbody(e, _):
      for u in range(hu):
        idx16 = idx_v[pl.ds((e * hu + u) * L, L)]
        plsc.addupdate_scatter(hist_v, [idx16], ones)
      return 0

    cp0.wait()
    lax.fori_loop(0, eh // (L * hu), hbody, 0)
    cp1.wait()
    lax.fori_loop(eh // (L * hu), ec // (L * hu), hbody, 0)

    pltpu.sync_copy(hist_v, shared.at[sid])
    plsc.subcore_barrier()

    pltpu.sync_copy(shared.at[:, pl.ds(sid * rs, rs)], red_v)

    def rbody(j, _):
      acc = red_v[0, pl.ds(j * L, L)]
      for k in range(1, NS):
        acc = acc + red_v[k, pl.ds(j * L, L)]
      acc_v[pl.ds(j * L, L)] = acc
      return 0

    lax.fori_loop(0, rs // L, rbody, 0)

    pltpu.sync_copy(acc_v, deg_hbm.at[pl.ds(cid * npad + sid * rs, rs)])

  return hist_kernel, npad


def _tc_matmul(x, W, b2, block_rows):
  n, dfeat = x.shape
  emb = W.shape[0]
  assert n % block_rows == 0

  def body(x_ref, w_ref, b_ref, o_ref):
    o_ref[...] = lax.dot_general(
        x_ref[...], w_ref[...],
        dimension_numbers=(((1,), (1,)), ((), ())),
        preferred_element_type=jnp.float32) + b_ref[...]

  return pl.pallas_call(
      body,
      grid=(n // block_rows,),
      in_specs=[
          pl.BlockSpec((block_rows, dfeat), lambda i: (i, 0)),
          pl.BlockSpec((emb, dfeat), lambda i: (0, 0)),
          pl.BlockSpec((1, emb), lambda i: (0, 0)),
      ],
      out_specs=pl.BlockSpec((block_rows, emb), lambda i: (i, 0)),
      out_shape=jax.ShapeDtypeStruct((n, emb), jnp.float32),
  )(x, W, b2)


def _sc_combine(deg2, degree_table, feat, n_nodes):
  npad = deg2.shape[0] // NC
  vocab, emb = degree_table.shape
  rs = npad // NW
  last = n_nodes - (NW - 1) * rs
  nch = 4
  gc = rs // nch
  assert rs % (nch * L) == 0 and gc <= 128 and last == gc and last % 8 == 0

  @functools.partial(
      pl.kernel,
      out_type=jax.ShapeDtypeStruct((n_nodes, emb), jnp.float32),
      mesh=_sc_mesh(),
      compiler_params=pltpu.CompilerParams(needs_layout_passes=False),
      scratch_types=[
          pltpu.VMEM((rs,), jnp.float32),
          pltpu.VMEM((rs,), jnp.float32),
          pltpu.VMEM((rs,), jnp.int32),
          pltpu.VMEM((rs, emb), jnp.float32),
          pltpu.VMEM_SHARED((vocab, emb), jnp.float32),
          pltpu.SemaphoreType.DMA,
          pltpu.SemaphoreType.DMA,
          pltpu.SemaphoreType.DMA,
          pltpu.SemaphoreType.DMA,
          pltpu.SemaphoreType.DMA,
          pltpu.SemaphoreType.DMA,
      ],
  )
  def combine_kernel(deg_hbm, tab_hbm, feat_hbm, out_hbm,
                     da_v, db_v, ii_v, acc_v, tab_sp, sem_d, sem_o,
                     sc0, sc1, sc2, sc3):
    scs = [sc0, sc1, sc2, sc3]
    cid = lax.axis_index("c")
    sid = lax.axis_index("s")
    wid = cid * NS + sid
    base = wid * rs
    is_last = wid == NW - 1

    cpa = pltpu.async_copy(deg_hbm.at[pl.ds(base, rs)], da_v, sem_d)
    cpb = pltpu.async_copy(deg_hbm.at[pl.ds(npad + base, rs)], db_v, sem_d)

    cp0 = pltpu.async_copy(
        feat_hbm.at[pl.ds(base, gc)], acc_v.at[pl.ds(0, gc)], scs[0])

    @pl.when(~is_last)
    def _():
      for k in range(1, nch):
        pltpu.async_copy(feat_hbm.at[pl.ds(base + k * gc, gc)],
                         acc_v.at[pl.ds(k * gc, gc)], scs[k])

    @pl.when(sid == 0)
    def _():
      pltpu.sync_copy(tab_hbm, tab_sp)

    cpa.wait()
    cpb.wait()

    def ibody(j, _):
      s = da_v[pl.ds(j * L, L)] + db_v[pl.ds(j * L, L)]
      ii_v[pl.ds(j * L, L)] = jnp.minimum(s.astype(jnp.int32), vocab - 1)
      return 0

    lax.fori_loop(0, rs // L, ibody, 0)

    plsc.subcore_barrier()

    cp0.wait()
    pltpu.async_copy(tab_sp.at[ii_v.at[pl.ds(0, gc)]],
                     acc_v.at[pl.ds(0, gc)], scs[0], add=True).wait()
    out0 = pltpu.async_copy(acc_v.at[pl.ds(0, gc)],
                            out_hbm.at[pl.ds(base, gc)], sem_o)

    @pl.when(~is_last)
    def _():
      for k in range(1, nch):
        pltpu.make_async_copy(feat_hbm.at[pl.ds(base + k * gc, gc)],
                              acc_v.at[pl.ds(k * gc, gc)], scs[k]).wait()
        pltpu.async_copy(tab_sp.at[ii_v.at[pl.ds(k * gc, gc)]],
                         acc_v.at[pl.ds(k * gc, gc)], scs[k], add=True).wait()
        pltpu.async_copy(acc_v.at[pl.ds(k * gc, gc)],
                         out_hbm.at[pl.ds(base + k * gc, gc)], sem_o)

    out0.wait()

    @pl.when(~is_last)
    def _():
      for k in range(1, nch):
        pltpu.make_async_copy(acc_v.at[pl.ds(k * gc, gc)],
                              out_hbm.at[pl.ds(base + k * gc, gc)],
                              sem_o).wait()

  return combine_kernel(deg2, degree_table, feat)


def kernel(x, edge_index, W, b, degree_table):
  n = x.shape[0]
  e_total = edge_index.shape[1]
  hist_kernel, _ = _sc_degree_histogram(e_total, n)
  deg2 = hist_kernel(edge_index.reshape(-1))
  feat = _tc_matmul(x, W, b.reshape(1, -1), block_rows=1000)
  return _sc_combine(deg2, degree_table, feat, n)


# speedup vs baseline: 11.2166x; 1.0324x over previous
import functools

import jax
import jax.numpy as jnp
from jax import lax
from jax.experimental import pallas as pl
from jax.experimental.pallas import tpu as pltpu
from jax.experimental.pallas import tpu_sc as plsc


NC = 2
NS = 16
L = 16
NW = NC * NS


def _sc_mesh():
  return plsc.VectorSubcoreMesh(core_axis_name="c", subcore_axis_name="s")


def _sc_degree_histogram(e_total, n_nodes):
  ec = 10240
  ec_last = e_total - (NW - 1) * ec
  assert ec_last > 0 and ec_last % 512 == 0
  npad = ((n_nodes + NS * L - 1) // (NS * L)) * (NS * L)
  rs = npad // NS

  @functools.partial(
      pl.kernel,
      out_type=jax.ShapeDtypeStruct((NC * npad,), jnp.float32),
      mesh=_sc_mesh(),
      compiler_params=pltpu.CompilerParams(needs_layout_passes=False),
      scratch_types=[
          pltpu.VMEM((2, ec), jnp.int32),
          pltpu.VMEM((npad,), jnp.float32),
          pltpu.VMEM_SHARED((NS, npad), jnp.float32),
          pltpu.VMEM((NS, rs), jnp.float32),
          pltpu.VMEM((rs,), jnp.float32),
          pltpu.SemaphoreType.DMA,
      ],
  )
  def hist_kernel(ei_hbm, deg_hbm, idx_v, hist_v, shared, red_v, acc_v, sem):
    cid = lax.axis_index("c")
    sid = lax.axis_index("s")
    wid = cid * NS + sid

    zeros = jnp.zeros((L,), jnp.float32)
    ones = jnp.ones((L,), jnp.float32)
    hu = 5

    def do_hist(csz):
      half = csz // 2
      assert half % (L * hu) == 0
      off = wid * ec
      cp0 = pltpu.async_copy(
          ei_hbm.at[:, pl.ds(off, half)], idx_v.at[:, pl.ds(0, half)], sem)
      cp1 = pltpu.async_copy(
          ei_hbm.at[:, pl.ds(off + half, half)],
          idx_v.at[:, pl.ds(half, half)], sem)

      zu = 4
      assert npad % (L * zu) == 0

      def zbody(i, _):
        for u in range(zu):
          hist_v[pl.ds((i * zu + u) * L, L)] = zeros
        return 0

      lax.fori_loop(0, npad // (L * zu), zbody, 0)

      def hbody(e, _):
        for u in range(hu):
          idx16 = idx_v[1, pl.ds((e * hu + u) * L, L)]
          plsc.addupdate_scatter(hist_v, [idx16], ones)
        return 0

      cp0.wait()
      lax.fori_loop(0, half // (L * hu), hbody, 0)
      cp1.wait()
      lax.fori_loop(half // (L * hu), csz // (L * hu), hbody, 0)

    @pl.when(wid < NW - 1)
    def _():
      do_hist(ec)

    @pl.when(wid == NW - 1)
    def _():
      do_hist(ec_last)

    pltpu.sync_copy(hist_v, shared.at[sid])
    plsc.subcore_barrier()

    pltpu.sync_copy(shared.at[:, pl.ds(sid * rs, rs)], red_v)

    def rbody(j, _):
      acc = red_v[0, pl.ds(j * L, L)]
      for k in range(1, NS):
        acc = acc + red_v[k, pl.ds(j * L, L)]
      acc_v[pl.ds(j * L, L)] = acc
      return 0

    lax.fori_loop(0, rs // L, rbody, 0)

    pltpu.sync_copy(acc_v, deg_hbm.at[pl.ds(cid * npad + sid * rs, rs)])

  return hist_kernel, npad


def _tc_matmul(x, W, b2, block_rows):
  n, dfeat = x.shape
  emb = W.shape[0]
  assert n % block_rows == 0

  def body(x_ref, w_ref, b_ref, o_ref):
    o_ref[...] = lax.dot_general(
        x_ref[...], w_ref[...],
        dimension_numbers=(((1,), (1,)), ((), ())),
        preferred_element_type=jnp.float32) + b_ref[...]

  return pl.pallas_call(
      body,
      grid=(n // block_rows,),
      in_specs=[
          pl.BlockSpec((block_rows, dfeat), lambda i: (i, 0)),
          pl.BlockSpec((emb, dfeat), lambda i: (0, 0)),
          pl.BlockSpec((1, emb), lambda i: (0, 0)),
      ],
      out_specs=pl.BlockSpec((block_rows, emb), lambda i: (i, 0)),
      out_shape=jax.ShapeDtypeStruct((n, emb), jnp.float32),
  )(x, W, b2)


def _sc_combine(deg2, degree_table, feat, n_nodes):
  npad = deg2.shape[0] // NC
  vocab, emb = degree_table.shape
  rs = npad // NW
  last = n_nodes - (NW - 1) * rs
  nch = 4
  gc = rs // nch
  assert rs % (nch * L) == 0 and gc <= 128 and last == gc and last % 8 == 0

  @functools.partial(
      pl.kernel,
      out_type=jax.ShapeDtypeStruct((n_nodes, emb), jnp.float32),
      mesh=_sc_mesh(),
      compiler_params=pltpu.CompilerParams(needs_layout_passes=False),
      scratch_types=[
          pltpu.VMEM((rs,), jnp.float32),
          pltpu.VMEM((rs,), jnp.float32),
          pltpu.VMEM((rs,), jnp.int32),
          pltpu.VMEM((rs, emb), jnp.float32),
          pltpu.VMEM_SHARED((vocab, emb), jnp.float32),
          pltpu.SemaphoreType.DMA,
          pltpu.SemaphoreType.DMA,
          pltpu.SemaphoreType.DMA,
          pltpu.SemaphoreType.DMA,
          pltpu.SemaphoreType.DMA,
          pltpu.SemaphoreType.DMA,
      ],
  )
  def combine_kernel(deg_hbm, tab_hbm, feat_hbm, out_hbm,
                     da_v, db_v, ii_v, acc_v, tab_sp, sem_d, sem_o,
                     sc0, sc1, sc2, sc3):
    scs = [sc0, sc1, sc2, sc3]
    cid = lax.axis_index("c")
    sid = lax.axis_index("s")
    wid = cid * NS + sid
    base = wid * rs
    is_last = wid == NW - 1

    cpa = pltpu.async_copy(deg_hbm.at[pl.ds(base, rs)], da_v, sem_d)
    cpb = pltpu.async_copy(deg_hbm.at[pl.ds(npad + base, rs)], db_v, sem_d)

    cp0 = pltpu.async_copy(
        feat_hbm.at[pl.ds(base, gc)], acc_v.at[pl.ds(0, gc)], scs[0])

    @pl.when(~is_last)
    def _():
      for k in range(1, nch):
        pltpu.async_copy(feat_hbm.at[pl.ds(base + k * gc, gc)],
                         acc_v.at[pl.ds(k * gc, gc)], scs[k])

    @pl.when(sid == 0)
    def _():
      pltpu.sync_copy(tab_hbm, tab_sp)

    cpa.wait()
    cpb.wait()

    def ibody(j, _):
      s = da_v[pl.ds(j * L, L)] + db_v[pl.ds(j * L, L)]
      ii_v[pl.ds(j * L, L)] = jnp.minimum(s.astype(jnp.int32), vocab - 1)
      return 0

    lax.fori_loop(0, rs // L, ibody, 0)

    plsc.subcore_barrier()

    cp0.wait()
    pltpu.async_copy(tab_sp.at[ii_v.at[pl.ds(0, gc)]],
                     acc_v.at[pl.ds(0, gc)], scs[0], add=True).wait()
    out0 = pltpu.async_copy(acc_v.at[pl.ds(0, gc)],
                            out_hbm.at[pl.ds(base, gc)], sem_o)

    @pl.when(~is_last)
    def _():
      for k in range(1, nch):
        pltpu.make_async_copy(feat_hbm.at[pl.ds(base + k * gc, gc)],
                              acc_v.at[pl.ds(k * gc, gc)], scs[k]).wait()
        pltpu.async_copy(tab_sp.at[ii_v.at[pl.ds(k * gc, gc)]],
                         acc_v.at[pl.ds(k * gc, gc)], scs[k], add=True).wait()
        pltpu.async_copy(acc_v.at[pl.ds(k * gc, gc)],
                         out_hbm.at[pl.ds(base + k * gc, gc)], sem_o)

    out0.wait()

    @pl.when(~is_last)
    def _():
      for k in range(1, nch):
        pltpu.make_async_copy(acc_v.at[pl.ds(k * gc, gc)],
                              out_hbm.at[pl.ds(base + k * gc, gc)],
                              sem_o).wait()

  return combine_kernel(deg2, degree_table, feat)


def kernel(x, edge_index, W, b, degree_table):
  n = x.shape[0]
  e_total = edge_index.shape[1]
  hist_kernel, _ = _sc_degree_histogram(e_total, n)
  deg2 = hist_kernel(edge_index)
  feat = _tc_matmul(x, W, b.reshape(1, -1), block_rows=2000)
  return _sc_combine(deg2, degree_table, feat, n)
